# Initial kernel scaffold; baseline (speedup 1.0000x reference)
import jax
import jax.numpy as jnp
from jax.experimental import pallas as pl


def kernel(lnc_indices, dis_indices, edge_lg, edge_gd, edge_ld, params):
    raise NotImplementedError("write your pallas kernel here")


# speedup vs baseline: 1.0093x; 1.0093x over previous
import jax
import jax.numpy as jnp
from jax.experimental import pallas as pl
from jax.experimental.pallas import tpu as pltpu

NUM_L = 10000
NUM_G = 10000
NUM_D = 10000
DIM = 128
HEADS = 8
HDIM = DIM // HEADS
LAYERS = 4
E = 160000
B = 16384
RELS = ["lg", "gl", "gd", "dg", "ld", "dl"]
TYPES = ["lnc", "gene", "dis"]


def _ln(x, s, b):
    mu = jnp.mean(x, axis=-1, keepdims=True)
    var = jnp.var(x, axis=-1, keepdims=True)
    return (x - mu) / jnp.sqrt(var + 1e-5) * s + b


def _attn(h_src, h_dst, src, dst, W, a_s, a_d, n_dst):
    ms = (h_src @ W).reshape(-1, HEADS, HDIM)
    s_score = jnp.sum(ms * a_s[None], axis=-1)
    d_score = jnp.sum((h_dst @ W).reshape(-1, HEADS, HDIM) * a_d[None], axis=-1)
    e = s_score[src] + d_score[dst]
    e = jax.nn.leaky_relu(e, 0.2)
    w = jnp.exp(e)
    den = jax.ops.segment_sum(w, dst, num_segments=n_dst)
    num = jax.ops.segment_sum(ms[src] * w[..., None], dst, num_segments=n_dst)
    out = num / (den[..., None] + 1e-16)
    return out.reshape(n_dst, DIM)


def _mlp_body(z_ref, w1_ref, b1_ref, w2_ref, b2_ref, o_ref):
    z = z_ref[...]
    h = jnp.maximum(z @ w1_ref[...] + b1_ref[...][None, :], 0.0)
    o_ref[...] = h @ w2_ref[...] + b2_ref[...][None, :]


def _mlp(z, w1, b1, w2, b2):
    grid = (B // 2048,)
    return pl.pallas_call(
        _mlp_body,
        grid=grid,
        in_specs=[
            pl.BlockSpec((2048, 2 * DIM), lambda i: (i, 0)),
            pl.BlockSpec((2 * DIM, DIM), lambda i: (0, 0)),
            pl.BlockSpec((DIM,), lambda i: (0,)),
            pl.BlockSpec((DIM, 1), lambda i: (0, 0)),
            pl.BlockSpec((1,), lambda i: (0,)),
        ],
        out_specs=pl.BlockSpec((2048, 1), lambda i: (i, 0)),
        out_shape=jax.ShapeDtypeStruct((B, 1), jnp.float32),
    )(z, w1, b1, w2, b2)


def kernel(lnc_indices, dis_indices, edge_lg, edge_gd, edge_ld, params):
    sizes = {"lnc": NUM_L, "gene": NUM_G, "dis": NUM_D}
    h = {"lnc": params["emb_lnc"] + params["pos_lnc"],
         "gene": params["emb_gene"] + params["pos_gene"],
         "dis": params["emb_dis"] + params["pos_dis"]}
    rels = [("lg", "lnc", "gene", edge_lg[0], edge_lg[1]),
            ("gl", "gene", "lnc", edge_lg[1], edge_lg[0]),
            ("gd", "gene", "dis", edge_gd[0], edge_gd[1]),
            ("dg", "dis", "gene", edge_gd[1], edge_gd[0]),
            ("ld", "lnc", "dis", edge_ld[0], edge_ld[1]),
            ("dl", "dis", "lnc", edge_ld[1], edge_ld[0])]
    for lp in params["layers"]:
        agg = {t: jnp.zeros((sizes[t], DIM), dtype=jnp.float32) for t in TYPES}
        for name, st, dt, src, dst in rels:
            agg[dt] = agg[dt] + _attn(h[st], h[dt], src, dst,
                                      lp["W_" + name], lp["as_" + name],
                                      lp["ad_" + name], sizes[dt])
        h = {t: _ln(h[t] + jax.nn.elu(agg[t] / 2.0), lp["ln_s_" + t], lp["ln_b_" + t])
             for t in TYPES}
    lnc_e = h["lnc"][lnc_indices]
    dis_e = h["dis"][dis_indices]
    z = jnp.concatenate([lnc_e, dis_e], axis=-1)
    return _mlp(z, params["mlp_W1"], params["mlp_b1"],
                params["mlp_W2"], params["mlp_b2"])[:, 0]


# speedup vs baseline: 24.3417x; 24.1173x over previous
import functools

import jax
import jax.numpy as jnp
from jax import lax
from jax.experimental import pallas as pl
from jax.experimental.pallas import tpu as pltpu
from jax.experimental.pallas import tpu_sc as plsc

N = 10000
DIM = 128
HEADS = 8
HH = 4
HD = 16
E = 160000
B = 16384
TROW = 128
NPAD = 10240
NT = 16
NC = 2
CH = 128
CHUNKS = 80
CG = CHUNKS // 8
EPT = CH * CHUNKS
EPAD = NT * EPT
ROWB = 1000
NB = N // ROWB

SRC_T = [0, 1, 1, 2, 0, 2]
DST_T = [1, 0, 2, 1, 2, 0]
REL_A = [1, 0, 2]
REL_B = [5, 3, 4]

_HIGH = jax.lax.Precision.HIGHEST


def _lut(r, table):
    out = jnp.int32(table[0])
    for k in range(1, len(table)):
        out = jnp.where(r == k, jnp.int32(table[k]), out)
    return out


def _k1_body(hs_ref, hd_ref, w_ref, wsf_ref, wdf_ref, tab_ref):
    m = jnp.dot(hs_ref[0], w_ref[0], precision=_HIGH)
    s8 = jnp.dot(hs_ref[0], wsf_ref[0], precision=_HIGH)
    d8 = jnp.dot(hd_ref[0], wdf_ref[0], precision=_HIGH)
    zpad = jnp.zeros((ROWB, TROW - 72), jnp.float32)
    for half in range(NC):
        tab_ref[0, half] = jnp.concatenate(
            [m[:, half * 64:(half + 1) * 64],
             s8[:, half * HH:(half + 1) * HH],
             d8[:, half * HH:(half + 1) * HH], zpad], axis=1)


def _k1(h_all, w6, wsf6, wdf6):
    return pl.pallas_call(
        _k1_body,
        grid=(6, NB),
        in_specs=[
            pl.BlockSpec((1, ROWB, DIM),
                         lambda r, i: (_lut(r, SRC_T), i, 0)),
            pl.BlockSpec((1, ROWB, DIM),
                         lambda r, i: (_lut(r, DST_T), i, 0)),
            pl.BlockSpec((1, DIM, DIM), lambda r, i: (r, 0, 0)),
            pl.BlockSpec((1, DIM, HEADS), lambda r, i: (r, 0, 0)),
            pl.BlockSpec((1, DIM, HEADS), lambda r, i: (r, 0, 0)),
        ],
        out_specs=[
            pl.BlockSpec((1, NC, ROWB, TROW), lambda r, i: (r, 0, i, 0)),
        ],
        out_shape=[
            jax.ShapeDtypeStruct((6, NC, NPAD, TROW), jnp.float32),
        ],
    )(h_all, h_all, w6, wsf6, wdf6)[0]


def _sc_body(tab_hbm, esrc_hbm, edst_hbm,
             nd_hbm,
             nd_s,
             srcc, dstc, srca, dsta, rows_v, drow_v, wt_v, semt, semd):
    cid = lax.axis_index("c")
    sid = lax.axis_index("s")
    lane = lax.iota(jnp.int32, 16)
    gidx0 = (lane & 3) * CH
    msk4 = jnp.where(lane < HH, 1.0, 0.0).astype(jnp.float32)
    zero16 = jnp.zeros((16,), jnp.float32)

    def per_rel(r, _):
        tbase = (r * NC + cid) * NPAD
        apr = NPAD // NT

        for e16 in range(CH):
            for c16 in range(TROW // 16):
                rows_v[e16, pl.ds(c16 * 16, 16)] = zero16
        for c in range(5):
            pltpu.sync_copy(rows_v,
                            nd_s.at[pl.ds(sid * apr + c * CH, CH)])
        plsc.subcore_barrier()

        def per_cgroup(cg, _):
            pltpu.sync_copy(esrc_hbm.at[r, sid, pl.ds(cg * 8, 8)], srcc)
            pltpu.sync_copy(edst_hbm.at[r, sid, pl.ds(cg * 8, 8)], dstc)
            for j in range(8):
                for g in range(CH // 16):
                    srca[j, pl.ds(g * 16, 16)] = (
                        srcc[j, pl.ds(g * 16, 16)] + tbase)
                    dsta[j, pl.ds(g * 16, 16)] = (
                        dstc[j, pl.ds(g * 16, 16)] + tbase)
            for j in range(8):
                cp_t = pltpu.async_copy(tab_hbm.at[srca.at[j]], rows_v, semt)
                cp_d = pltpu.async_copy(tab_hbm.at[dsta.at[j]], drow_v, semd)
                cp_t.wait()
                cp_d.wait()

                for g in range(CH // 16):
                    eidx = lane + (g * 16)
                    for h in range(HH):
                        s = plsc.load_gather(
                            rows_v,
                            [eidx, jnp.full((16,), 64 + h, jnp.int32)])
                        d = plsc.load_gather(
                            drow_v,
                            [eidx, jnp.full((16,), 68 + h, jnp.int32)])
                        x = s + d
                        x = jnp.maximum(x, x * 0.2)
                        wt_v[pl.ds(h * CH + g * 16, 16)] = jnp.exp(x)

                def edge_body(e, _):
                    w16 = plsc.load_gather(wt_v, [gidx0 + e]) * msk4
                    for h in range(HH):
                        rows_v[e, pl.ds(h * HD, HD)] = (
                            rows_v[e, pl.ds(h * HD, HD)] * w16[h])
                    rows_v[e, pl.ds(64, 16)] = w16
                    return 0

                lax.fori_loop(0, CH, edge_body, 0, unroll=4)

                pltpu.sync_copy(rows_v, nd_s.at[dstc.at[j]], add=True)
            return 0

        lax.fori_loop(0, CG, per_cgroup, 0)
        plsc.subcore_barrier()

        for c in range(5):
            off = sid * apr + c * CH
            pltpu.sync_copy(nd_s.at[pl.ds(off, CH)], rows_v)
            pltpu.sync_copy(rows_v, nd_hbm.at[r, cid, pl.ds(off, CH)])
        return 0

    lax.fori_loop(0, 6, per_rel, 0)


def _sc_attn(tab_flat, esrc, edst):
    mesh = plsc.VectorSubcoreMesh(core_axis_name="c", subcore_axis_name="s")
    f = functools.partial(
        pl.kernel,
        out_type=jax.ShapeDtypeStruct((6, NC, NPAD, TROW), jnp.float32),
        mesh=mesh,
        compiler_params=pltpu.CompilerParams(needs_layout_passes=False),
        scratch_types=[
            pltpu.VMEM_SHARED((NPAD, TROW), jnp.float32),
            pltpu.VMEM((8, CH), jnp.int32),
            pltpu.VMEM((8, CH), jnp.int32),
            pltpu.VMEM((8, CH), jnp.int32),
            pltpu.VMEM((8, CH), jnp.int32),
            pltpu.VMEM((CH, TROW), jnp.float32),
            pltpu.VMEM((CH, TROW), jnp.float32),
            pltpu.VMEM((HH * CH,), jnp.float32),
            pltpu.SemaphoreType.DMA,
            pltpu.SemaphoreType.DMA,
        ],
    )(_sc_body)
    return f(tab_flat, esrc, edst)


def _k2_body(h_ref, a_ref, b_ref, ex_ref, s_ref, bb_ref, out_ref):
    ex = ex_ref[...]

    def half_agg(nd_ref):
        parts = []
        for half in range(NC):
            nd = nd_ref[0, half]
            dden = jnp.dot(nd[:, 64:80], ex, precision=_HIGH)
            parts.append(nd[:, :64] / (dden + 1e-16))
        return jnp.concatenate(parts, axis=-1)

    agg = half_agg(a_ref) + half_agg(b_ref)
    x = agg * 0.5
    y = h_ref[0] + jnp.where(x > 0.0, x, jnp.exp(x) - 1.0)
    mu = jnp.mean(y, axis=-1, keepdims=True)
    yc = y - mu
    var = jnp.mean(yc * yc, axis=-1, keepdims=True)
    out_ref[0] = yc * jax.lax.rsqrt(var + 1e-5) * s_ref[0] + bb_ref[0]


def _k2(h_all, nd, ex, ln_s, ln_b):
    return pl.pallas_call(
        _k2_body,
        grid=(3, NB),
        in_specs=[
            pl.BlockSpec((1, ROWB, DIM), lambda t, i: (t, i, 0)),
            pl.BlockSpec((1, NC, ROWB, TROW),
                         lambda t, i: (_lut(t, REL_A), 0, i, 0)),
            pl.BlockSpec((1, NC, ROWB, TROW),
                         lambda t, i: (_lut(t, REL_B), 0, i, 0)),
            pl.BlockSpec((16, 64), lambda t, i: (0, 0)),
            pl.BlockSpec((1, 1, DIM), lambda t, i: (t, 0, 0)),
            pl.BlockSpec((1, 1, DIM), lambda t, i: (t, 0, 0)),
        ],
        out_specs=pl.BlockSpec((1, ROWB, DIM), lambda t, i: (t, i, 0)),
        out_shape=jax.ShapeDtypeStruct((3, N, DIM), jnp.float32),
    )(h_all, nd, nd, ex, ln_s.reshape(3, 1, DIM), ln_b.reshape(3, 1, DIM))


def _gather_body(hl_hbm, hd_hbm, li_hbm, di_hbm, z_hbm, idx_v, rows_v, sem):
    cid = lax.axis_index("c")
    sid = lax.axis_index("s")
    wid = sid * NC + cid
    bpw = B // (NC * NT)
    base = wid * bpw
    pltpu.sync_copy(li_hbm.at[pl.ds(base, bpw)], idx_v)
    pltpu.async_copy(hl_hbm.at[idx_v], rows_v, sem).wait()
    pltpu.sync_copy(rows_v, z_hbm.at[0, pl.ds(base, bpw)])
    pltpu.sync_copy(di_hbm.at[pl.ds(base, bpw)], idx_v)
    pltpu.async_copy(hd_hbm.at[idx_v], rows_v, sem).wait()
    pltpu.sync_copy(rows_v, z_hbm.at[1, pl.ds(base, bpw)])


def _pair_gather(h_lnc, h_dis, lnc_idx, dis_idx):
    mesh = plsc.VectorSubcoreMesh(core_axis_name="c", subcore_axis_name="s")
    bpw = B // (NC * NT)
    f = functools.partial(
        pl.kernel,
        out_type=jax.ShapeDtypeStruct((2, B, DIM), jnp.float32),
        mesh=mesh,
        compiler_params=pltpu.CompilerParams(needs_layout_passes=False),
        scratch_types=[
            pltpu.VMEM((bpw,), jnp.int32),
            pltpu.VMEM((bpw, DIM), jnp.float32),
            pltpu.SemaphoreType.DMA,
        ],
    )(_gather_body)
    return f(h_lnc, h_dis, lnc_idx, dis_idx)


def _mlp_body(z0_ref, z1_ref, w1a_ref, w1b_ref, b1_ref, w2_ref, b2_ref,
              o_ref):
    m = (jnp.dot(z0_ref[0], w1a_ref[...], precision=_HIGH)
         + jnp.dot(z1_ref[0], w1b_ref[...], precision=_HIGH)
         + b1_ref[...][None, :])
    m = jnp.maximum(m, 0.0)
    o_ref[...] = jnp.dot(m, w2_ref[...], precision=_HIGH) + b2_ref[...][None, :]


def _mlp(z, w1, b1, w2, b2):
    blk = 2048
    return pl.pallas_call(
        _mlp_body,
        grid=(B // blk,),
        in_specs=[
            pl.BlockSpec((1, blk, DIM), lambda i: (0, i, 0)),
            pl.BlockSpec((1, blk, DIM), lambda i: (1, i, 0)),
            pl.BlockSpec((DIM, DIM), lambda i: (0, 0)),
            pl.BlockSpec((DIM, DIM), lambda i: (0, 0)),
            pl.BlockSpec((DIM,), lambda i: (0,)),
            pl.BlockSpec((DIM, 1), lambda i: (0, 0)),
            pl.BlockSpec((1,), lambda i: (0,)),
        ],
        out_specs=pl.BlockSpec((blk, 1), lambda i: (i, 0)),
        out_shape=jax.ShapeDtypeStruct((B, 1), jnp.float32),
    )(z, z, w1[:DIM], w1[DIM:], b1, w2, b2)


def _pad_edges(src, dst):
    pad = EPAD - E
    src_p = jnp.concatenate([src, jnp.zeros((pad,), jnp.int32)])
    dst_p = jnp.concatenate([dst, jnp.full((pad,), N, jnp.int32)])
    return src_p.reshape(NT, CHUNKS, CH), dst_p.reshape(NT, CHUNKS, CH)


def kernel(lnc_indices, dis_indices, edge_lg, edge_gd, edge_ld, params):
    pairs = [(edge_lg[0], edge_lg[1]), (edge_lg[1], edge_lg[0]),
             (edge_gd[0], edge_gd[1]), (edge_gd[1], edge_gd[0]),
             (edge_ld[0], edge_ld[1]), (edge_ld[1], edge_ld[0])]
    es, ed = [], []
    for s, d in pairs:
        sp, dp = _pad_edges(s, d)
        es.append(sp)
        ed.append(dp)
    esrc = jnp.stack(es)
    edst = jnp.stack(ed)


    eye4 = jnp.eye(HH, dtype=jnp.float32)
    ex = jnp.concatenate(
        [jnp.repeat(eye4, HD, axis=1),
         jnp.zeros((16 - HH, HH * HD), jnp.float32)], axis=0)

    eye8 = jnp.eye(HEADS, dtype=jnp.float32)
    layer_w = []
    for lp in params["layers"]:
        w6 = jnp.stack([lp["W_" + r] for r in
                        ["lg", "gl", "gd", "dg", "ld", "dl"]])
        asf, adf = [], []
        for r in ["lg", "gl", "gd", "dg", "ld", "dl"]:
            a_s = (lp["as_" + r][:, :, None] * eye8[:, None, :]) \
                .reshape(DIM, HEADS)
            a_d = (lp["ad_" + r][:, :, None] * eye8[:, None, :]) \
                .reshape(DIM, HEADS)
            asf.append(lp["W_" + r] @ a_s)
            adf.append(lp["W_" + r] @ a_d)
        wsf6 = jnp.stack(asf)
        wdf6 = jnp.stack(adf)
        ln_s = jnp.stack([lp["ln_s_" + t] for t in ["lnc", "gene", "dis"]])
        ln_b = jnp.stack([lp["ln_b_" + t] for t in ["lnc", "gene", "dis"]])
        layer_w.append((w6, wsf6, wdf6, ln_s, ln_b))

    h_all = jnp.stack([params["emb_lnc"] + params["pos_lnc"],
                       params["emb_gene"] + params["pos_gene"],
                       params["emb_dis"] + params["pos_dis"]])

    for w6, wsf6, wdf6, ln_s, ln_b in layer_w:
        tab = _k1(h_all, w6, wsf6, wdf6)
        nd = _sc_attn(tab.reshape(6 * NC * NPAD, TROW), esrc, edst)
        h_all = _k2(h_all, nd, ex, ln_s, ln_b)

    z = _pair_gather(h_all[0], h_all[2], lnc_indices, dis_indices)
    return _mlp(z, params["mlp_W1"], params["mlp_b1"],
                params["mlp_W2"], params["mlp_b2"])[:, 0]
